# baseline (speedup 1.0000x reference)
import functools

import jax
import jax.numpy as jnp
from jax.experimental import pallas as pl
from jax.experimental.pallas import tpu as pltpu
from jax.experimental.pallas import tpu_sc as plsc

_EMBED_DIM = 64
_PAD_DIM = 128
_NUM_CODES = 1024
_B = 16
_S = 576
_N_TOK = _B * _S
_TILE = 3072
_STEPS = _N_TOK // _TILE

_NC = 2
_NS = 16
_NW = _NC * _NS
_NWORK = 24
_RPW = 3


def _idx_body(z_ref, w_ref, idx_ref, wdup_ref, wn_ref):
    @pl.when(pl.program_id(0) == 0)
    def _init():
        w = w_ref[...]
        wn_ref[...] = w / jnp.maximum(
            jnp.sqrt(jnp.sum(w * w, axis=1, keepdims=True)), 1e-12)
        wdup_ref[:, :_EMBED_DIM] = w
        wdup_ref[:, _EMBED_DIM:] = w

    zt = z_ref[...]
    zn = zt / jnp.maximum(
        jnp.sqrt(jnp.sum(zt * zt, axis=1, keepdims=True)), 1e-12)
    scores = jax.lax.dot_general(
        zn, wn_ref[...], (((1,), (1,)), ((), ())),
        preferred_element_type=jnp.float32)
    m = jnp.max(scores, axis=1, keepdims=True)
    ids = jax.lax.broadcasted_iota(
        jnp.int32, scores.shape, 1).astype(jnp.float32)
    idx = jnp.min(jnp.where(scores == m, ids, jnp.float32(4096.0)), axis=1)
    i = pl.program_id(0)
    idx_ref[...] = idx.astype(jnp.int32).reshape(_TILE // 128, 128)


def _gather_body(wdup_hbm, idx_hbm, out_hbm, idx_v, rows_v, sem):
    wid = jax.lax.axis_index("s") * _NC + jax.lax.axis_index("c")

    @pl.when(wid < _NWORK)
    def _work():
        pltpu.sync_copy(idx_hbm.at[pl.ds(wid * _RPW, _RPW), :], idx_v)
        for j in range(_RPW):
            pltpu.async_copy(
                wdup_hbm.at[idx_v.at[j]], rows_v.at[j], sem).wait()
            pltpu.sync_copy(
                rows_v.at[j, :, pl.ds(0, _EMBED_DIM)],
                out_hbm.at[pl.ds((wid * _RPW + j) * 128, 128)])


_sc_gather = pl.kernel(
    _gather_body,
    out_type=jax.ShapeDtypeStruct((_N_TOK, _EMBED_DIM), jnp.float32),
    mesh=plsc.VectorSubcoreMesh(
        core_axis_name="c", subcore_axis_name="s",
        num_cores=_NC, num_subcores=_NS),
    scratch_types=[
        pltpu.VMEM((_RPW, 128), jnp.int32),
        pltpu.VMEM((_RPW, 128, _PAD_DIM), jnp.float32),
        pltpu.SemaphoreType.DMA,
    ],
    compiler_params=pltpu.CompilerParams(use_tc_tiling_on_sc=False),
)


@jax.jit
def kernel(z, W):
    z2 = z.reshape(_N_TOK, _EMBED_DIM)
    idx, wdup = pl.pallas_call(
        _idx_body,
        grid=(_STEPS,),
        in_specs=[
            pl.BlockSpec((_TILE, _EMBED_DIM), lambda i: (i, 0)),
            pl.BlockSpec((_NUM_CODES, _EMBED_DIM), lambda i: (0, 0)),
        ],
        out_specs=[
            pl.BlockSpec((_TILE // 128, 128), lambda i: (i, 0)),
            pl.BlockSpec((_NUM_CODES, _PAD_DIM), lambda i: (0, 0)),
        ],
        out_shape=[
            jax.ShapeDtypeStruct((_N_TOK // 128, 128), jnp.int32),
            jax.ShapeDtypeStruct((_NUM_CODES, _PAD_DIM), jnp.float32),
        ],
        scratch_shapes=[pltpu.VMEM((_NUM_CODES, _EMBED_DIM), jnp.float32)],
    )(z2, W)
    return _sc_gather(wdup, idx).reshape(_B, _S, _EMBED_DIM)


# speedup vs baseline: 1.0100x; 1.0100x over previous
import functools

import jax
import jax.numpy as jnp
from jax.experimental import pallas as pl
from jax.experimental.pallas import tpu as pltpu
from jax.experimental.pallas import tpu_sc as plsc

_EMBED_DIM = 64
_PAD_DIM = 128
_NUM_CODES = 1024
_B = 16
_S = 576
_N_TOK = _B * _S
_TILE = 3072
_STEPS = _N_TOK // _TILE

_NC = 2
_NS = 16
_NW = _NC * _NS
_RPW = 3


def _idx_body(z_ref, w_ref, idx_ref, wdup_ref, wn_ref):
    @pl.when(pl.program_id(0) == 0)
    def _init():
        w = w_ref[...]
        wn_ref[...] = w / jnp.maximum(
            jnp.sqrt(jnp.sum(w * w, axis=1, keepdims=True)), 1e-12)
        wdup_ref[:, :_EMBED_DIM] = w
        wdup_ref[:, _EMBED_DIM:] = w

    zt = z_ref[...]
    zn = zt / jnp.maximum(
        jnp.sqrt(jnp.sum(zt * zt, axis=1, keepdims=True)), 1e-12)
    scores = jax.lax.dot_general(
        zn, wn_ref[...], (((1,), (1,)), ((), ())),
        preferred_element_type=jnp.float32)
    m = jnp.max(scores, axis=1, keepdims=True)
    ids = jax.lax.broadcasted_iota(
        jnp.int32, scores.shape, 1).astype(jnp.float32)
    idx = jnp.min(jnp.where(scores == m, ids, jnp.float32(4096.0)), axis=1)
    i = pl.program_id(0)
    idx_ref[...] = idx.astype(jnp.int32).reshape(_TILE // 128, 128)


def _gather_body(wdup_hbm, idx_hbm, out_hbm, idx_v, rows_v, sems):
    wid = jax.lax.axis_index("s") * _NC + jax.lax.axis_index("c")
    rows = [wid, wid + _NW, wid + 2 * _NW]

    copies = []
    for j, r in enumerate(rows):
        extra = j == 2
        @pl.when(wid < 8 if extra else wid >= 0)
        def _load(r=r, j=j):
            pltpu.sync_copy(idx_hbm.at[pl.ds(r, 1), :], idx_v.at[pl.ds(j, 1)])
    for j, r in enumerate(rows):
        extra = j == 2
        @pl.when(wid < 8 if extra else wid >= 0)
        def _fire(r=r, j=j):
            pltpu.async_copy(wdup_hbm.at[idx_v.at[j]], rows_v.at[j], sems[j])
    for j, r in enumerate(rows):
        extra = j == 2
        @pl.when(wid < 8 if extra else wid >= 0)
        def _drain(r=r, j=j):
            pltpu.make_async_copy(
                wdup_hbm.at[idx_v.at[j]], rows_v.at[j], sems[j]).wait()
            pltpu.sync_copy(
                rows_v.at[j, :, pl.ds(0, _EMBED_DIM)],
                out_hbm.at[pl.ds(r * 128, 128)])


_sc_gather = pl.kernel(
    _gather_body,
    out_type=jax.ShapeDtypeStruct((_N_TOK, _EMBED_DIM), jnp.float32),
    mesh=plsc.VectorSubcoreMesh(
        core_axis_name="c", subcore_axis_name="s",
        num_cores=_NC, num_subcores=_NS),
    scratch_types=[
        pltpu.VMEM((_RPW, 128), jnp.int32),
        pltpu.VMEM((_RPW, 128, _PAD_DIM), jnp.float32),
        [pltpu.SemaphoreType.DMA] * _RPW,
    ],
    compiler_params=pltpu.CompilerParams(use_tc_tiling_on_sc=False),
)


@jax.jit
def kernel(z, W):
    z2 = z.reshape(_N_TOK, _EMBED_DIM)
    idx, wdup = pl.pallas_call(
        _idx_body,
        grid=(_STEPS,),
        in_specs=[
            pl.BlockSpec((_TILE, _EMBED_DIM), lambda i: (i, 0)),
            pl.BlockSpec((_NUM_CODES, _EMBED_DIM), lambda i: (0, 0)),
        ],
        out_specs=[
            pl.BlockSpec((_TILE // 128, 128), lambda i: (i, 0)),
            pl.BlockSpec((_NUM_CODES, _PAD_DIM), lambda i: (0, 0)),
        ],
        out_shape=[
            jax.ShapeDtypeStruct((_N_TOK // 128, 128), jnp.int32),
            jax.ShapeDtypeStruct((_NUM_CODES, _PAD_DIM), jnp.float32),
        ],
        scratch_shapes=[pltpu.VMEM((_NUM_CODES, _EMBED_DIM), jnp.float32)],
    )(z2, W)
    return _sc_gather(wdup, idx).reshape(_B, _S, _EMBED_DIM)
